# baseline (speedup 1.0000x reference)
import functools
import numpy as np
import jax
import jax.numpy as jnp
from jax import lax
from jax.experimental import pallas as pl
from jax.experimental.pallas import tpu as pltpu
from jax.experimental.pallas import tpu_sc as plsc

NUM_NODES = 100000
EMBED_DIM = 128
BATCH = 10000
NUM_NEIGHBORS = 32

_NC, _NS = 2, 16
_NW = _NC * _NS
_B_PAD = 10240
_B_PER_W = _B_PAD // _NW
_CHUNK_IDX = 128
_B_PER_CHUNK = _CHUNK_IDX // NUM_NEIGHBORS
_CHUNKS_PER_W = _B_PER_W // _B_PER_CHUNK
_NBUF = 2

_COL_PERM = np.concatenate(
    [32 * g + np.arange(32).reshape(2, 16).T.reshape(-1) for g in range(4)])


def _tec_body(theta_hbm, idx_hbm, out_hbm, idx_v, rows0, rows1,
              out_v, sem0, sem1):
    wid = lax.axis_index("s") * _NC + lax.axis_index("c")
    pltpu.sync_copy(idx_hbm.at[pl.ds(wid * _CHUNKS_PER_W, _CHUNKS_PER_W)], idx_v)
    bufs = (rows0, rows1)
    sems = (sem0, sem1)

    def start(c, b):
        pltpu.async_copy(theta_hbm.at[idx_v.at[c]], bufs[b], sems[b])

    def reduce(c, b):
        rows = bufs[b]
        for bb in range(_B_PER_CHUNK):
            ob = c * _B_PER_CHUNK + bb
            for g in range(EMBED_DIM // 32):
                los, his = [], []
                for k in range(NUM_NEIGHBORS):
                    e, o = plsc.unpack(
                        rows[bb * NUM_NEIGHBORS + k, pl.ds(g * 32, 32)],
                        format=plsc.PackFormat.INTERLEAVED)
                    los.append(e)
                    his.append(o)
                while len(los) > 1:
                    los = [los[i] + los[i + 1] for i in range(0, len(los), 2)]
                    his = [his[i] + his[i + 1] for i in range(0, len(his), 2)]
                out_v[ob, pl.ds(g * 32, 16)] = los[0] * (1.0 / NUM_NEIGHBORS)
                out_v[ob, pl.ds(g * 32 + 16, 16)] = his[0] * (1.0 / NUM_NEIGHBORS)

    for b in range(_NBUF):
        start(b, b)

    def step(j, _):
        for b in range(_NBUF):
            c = j * _NBUF + b
            pltpu.make_async_copy(theta_hbm.at[idx_v.at[c]], bufs[b],
                                  sems[b]).wait()
            reduce(c, b)

            @pl.when(c + _NBUF < _CHUNKS_PER_W)
            def _():
                start(c + _NBUF, b)
        return ()

    lax.fori_loop(0, _CHUNKS_PER_W // _NBUF, step, (), unroll=False)
    pltpu.sync_copy(out_v, out_hbm.at[pl.ds(wid * _B_PER_W, _B_PER_W)])


@jax.jit
def kernel(node_ids, neighbor_lists, theta):
    del node_ids
    theta_bf = theta.astype(jnp.bfloat16)
    nbr = jnp.zeros((_B_PAD, NUM_NEIGHBORS), jnp.int32)
    nbr = nbr.at[:BATCH].set(neighbor_lists)
    idx = nbr.reshape(_B_PAD * NUM_NEIGHBORS // _CHUNK_IDX, _CHUNK_IDX)

    mesh = plsc.VectorSubcoreMesh(core_axis_name="c", subcore_axis_name="s")
    out = pl.kernel(
        _tec_body,
        out_type=jax.ShapeDtypeStruct((_B_PAD, EMBED_DIM), jnp.float32),
        mesh=mesh,
        compiler_params=pltpu.CompilerParams(needs_layout_passes=False,
                                             use_tc_tiling_on_sc=False),
        scratch_types=[
            pltpu.VMEM((_CHUNKS_PER_W, _CHUNK_IDX), jnp.int32),
            pltpu.VMEM((_CHUNK_IDX, EMBED_DIM), jnp.bfloat16),
            pltpu.VMEM((_CHUNK_IDX, EMBED_DIM), jnp.bfloat16),
            pltpu.VMEM((_B_PER_W, EMBED_DIM), jnp.float32),
            pltpu.SemaphoreType.DMA,
            pltpu.SemaphoreType.DMA,
        ],
    )(theta_bf, idx)
    return out[:BATCH, _COL_PERM]


# speedup vs baseline: 3.4087x; 3.4087x over previous
import functools
import numpy as np
import jax
import jax.numpy as jnp
from jax import lax
from jax.experimental import pallas as pl
from jax.experimental.pallas import tpu as pltpu
from jax.experimental.pallas import tpu_sc as plsc

NUM_NODES = 100000
EMBED_DIM = 128
BATCH = 10000
NUM_NEIGHBORS = 32

_NC, _NS = 2, 16
_NW = _NC * _NS
_CHUNK_IDX = 128
_B_PER_CHUNK = _CHUNK_IDX // NUM_NEIGHBORS
_N_CHUNKS = BATCH * NUM_NEIGHBORS // _CHUNK_IDX
_NBUF = 2

_T0 = 970
_T1 = _N_CHUNKS - _T0
_Q0, _R0 = _T0 // _NS, _T0 % _NS
_Q1, _R1 = _T1 // _NS, _T1 % _NS
_IDX_ROWS = max(_Q0, _Q1) + 1

_COL_PERM = np.concatenate(
    [32 * g + np.arange(32).reshape(2, 16).T.reshape(-1) for g in range(4)])


def _tec_body(theta_hbm, idx_hbm, out_hbm, idx_v, rows0, rows1, oc0, oc1,
              gsem0, gsem1, ssem0, ssem1):
    cid = lax.axis_index("c")
    sid = lax.axis_index("s")
    is0 = cid == 0
    side_start = jnp.where(is0, 0, _T0)
    q = jnp.where(is0, _Q0, _Q1)
    r = jnp.where(is0, _R0, _R1)
    start_chunk = side_start + q * sid + jnp.minimum(sid, r)
    extra = sid < r
    n_chunks = q + extra.astype(jnp.int32)

    @pl.when(is0)
    def _():
        pltpu.sync_copy(idx_hbm.at[pl.ds(start_chunk, _Q0)],
                        idx_v.at[pl.ds(0, _Q0)])

    @pl.when(jnp.logical_not(is0))
    def _():
        pltpu.sync_copy(idx_hbm.at[pl.ds(start_chunk, _Q1)],
                        idx_v.at[pl.ds(0, _Q1)])

    @pl.when(extra)
    def _():
        for qq in (_Q0, _Q1):
            @pl.when(q == qq)
            def _(qq=qq):
                pltpu.sync_copy(idx_hbm.at[pl.ds(start_chunk + qq, 1)],
                                idx_v.at[pl.ds(qq, 1)])

    bufs = (rows0, rows1)
    outs = (oc0, oc1)
    gsems = (gsem0, gsem1)
    ssems = (ssem0, ssem1)

    def start(c, b):
        pltpu.async_copy(theta_hbm.at[idx_v.at[c]], bufs[b], gsems[b])

    def reduce(b):
        rows = bufs[b]
        for bb in range(_B_PER_CHUNK):
            for g in range(EMBED_DIM // 32):
                los, his = [], []
                for k in range(NUM_NEIGHBORS):
                    e, o = plsc.unpack(
                        rows[bb * NUM_NEIGHBORS + k, pl.ds(g * 32, 32)],
                        format=plsc.PackFormat.INTERLEAVED)
                    los.append(e)
                    his.append(o)
                while len(los) > 1:
                    los = [los[i] + los[i + 1] for i in range(0, len(los), 2)]
                    his = [his[i] + his[i + 1] for i in range(0, len(his), 2)]
                outs[b][bb, pl.ds(g * 32, 16)] = los[0] * (1.0 / NUM_NEIGHBORS)
                outs[b][bb, pl.ds(g * 32 + 16, 16)] = (
                    his[0] * (1.0 / NUM_NEIGHBORS))

    for b in range(_NBUF):
        @pl.when(b < n_chunks)
        def _(b=b):
            start(b, b)

    def step(c, _):
        b = lax.rem(c, _NBUF)
        for bs in range(_NBUF):
            @pl.when(b == bs)
            def _(bs=bs):
                pltpu.make_async_copy(theta_hbm.at[idx_v.at[c]], bufs[bs],
                                      gsems[bs]).wait()

                @pl.when(c >= _NBUF)
                def _():
                    pltpu.make_async_copy(
                        outs[bs],
                        out_hbm.at[pl.ds(0, _B_PER_CHUNK)],
                        ssems[bs]).wait()

                reduce(bs)
                pltpu.async_copy(
                    outs[bs],
                    out_hbm.at[pl.ds((start_chunk + c) * _B_PER_CHUNK,
                                     _B_PER_CHUNK)],
                    ssems[bs])

                @pl.when(c + _NBUF < n_chunks)
                def _():
                    start(c + _NBUF, bs)
        return ()

    lax.fori_loop(0, n_chunks, step, (), unroll=False)
    for b in range(_NBUF):
        @pl.when(b < n_chunks)
        def _(b=b):
            pltpu.make_async_copy(outs[b],
                                  out_hbm.at[pl.ds(0, _B_PER_CHUNK)],
                                  ssems[b]).wait()


@jax.jit
def kernel(node_ids, neighbor_lists, theta):
    del node_ids
    theta_bf = theta.astype(jnp.bfloat16)[:, _COL_PERM]
    idx = neighbor_lists.reshape(_N_CHUNKS, _CHUNK_IDX)

    mesh = plsc.VectorSubcoreMesh(core_axis_name="c", subcore_axis_name="s")
    out = pl.kernel(
        _tec_body,
        out_type=jax.ShapeDtypeStruct((BATCH, EMBED_DIM), jnp.float32),
        mesh=mesh,
        compiler_params=pltpu.CompilerParams(needs_layout_passes=False,
                                             use_tc_tiling_on_sc=False),
        scratch_types=[
            pltpu.VMEM((_IDX_ROWS, _CHUNK_IDX), jnp.int32),
            pltpu.VMEM((_CHUNK_IDX, EMBED_DIM), jnp.bfloat16),
            pltpu.VMEM((_CHUNK_IDX, EMBED_DIM), jnp.bfloat16),
            pltpu.VMEM((_B_PER_CHUNK, EMBED_DIM), jnp.float32),
            pltpu.VMEM((_B_PER_CHUNK, EMBED_DIM), jnp.float32),
            pltpu.SemaphoreType.DMA,
            pltpu.SemaphoreType.DMA,
            pltpu.SemaphoreType.DMA,
            pltpu.SemaphoreType.DMA,
        ],
    )(theta_bf, idx)
    return out


# speedup vs baseline: 3.5237x; 1.0337x over previous
import functools
import numpy as np
import jax
import jax.numpy as jnp
from jax import lax
from jax.experimental import pallas as pl
from jax.experimental.pallas import tpu as pltpu
from jax.experimental.pallas import tpu_sc as plsc

NUM_NODES = 100000
EMBED_DIM = 128
BATCH = 10000
NUM_NEIGHBORS = 32

_NC, _NS = 2, 16
_NW = _NC * _NS
_CHUNK_IDX = 128
_B_PER_CHUNK = _CHUNK_IDX // NUM_NEIGHBORS
_N_CHUNKS = BATCH * NUM_NEIGHBORS // _CHUNK_IDX
_CHUNKS_LO = _N_CHUNKS // _NW
_N_HI = _N_CHUNKS - _CHUNKS_LO * _NW
_NBUF = 2

_COL_PERM = np.concatenate(
    [32 * g + np.arange(32).reshape(2, 16).T.reshape(-1) for g in range(4)])


def _tec_body(theta_hbm, idx_hbm, out_hbm, idx_v, rows0, rows1, oc0, oc1,
              gsem0, gsem1, ssem0, ssem1):
    wid = lax.axis_index("s") * _NC + lax.axis_index("c")
    is_hi = wid < _N_HI
    start_chunk = jnp.where(is_hi, (_CHUNKS_LO + 1) * wid,
                            (_CHUNKS_LO + 1) * _N_HI
                            + _CHUNKS_LO * (wid - _N_HI))
    n_chunks = jnp.where(is_hi, _CHUNKS_LO + 1, _CHUNKS_LO)

    pltpu.sync_copy(idx_hbm.at[pl.ds(start_chunk, _CHUNKS_LO)],
                    idx_v.at[pl.ds(0, _CHUNKS_LO)])

    @pl.when(is_hi)
    def _():
        pltpu.sync_copy(idx_hbm.at[pl.ds(start_chunk + _CHUNKS_LO, 1)],
                        idx_v.at[pl.ds(_CHUNKS_LO, 1)])

    bufs = (rows0, rows1)
    outs = (oc0, oc1)
    gsems = (gsem0, gsem1)
    ssems = (ssem0, ssem1)

    def start(c, b):
        pltpu.async_copy(theta_hbm.at[idx_v.at[c]], bufs[b], gsems[b])

    def reduce(b):
        rows = bufs[b]
        for bb in range(_B_PER_CHUNK):
            for g in range(EMBED_DIM // 32):
                los, his = [], []
                for k in range(NUM_NEIGHBORS):
                    e, o = plsc.unpack(
                        rows[bb * NUM_NEIGHBORS + k, pl.ds(g * 32, 32)],
                        format=plsc.PackFormat.INTERLEAVED)
                    los.append(e)
                    his.append(o)
                while len(los) > 1:
                    los = [los[i] + los[i + 1] for i in range(0, len(los), 2)]
                    his = [his[i] + his[i + 1] for i in range(0, len(his), 2)]
                outs[b][bb, pl.ds(g * 32, 16)] = los[0] * (1.0 / NUM_NEIGHBORS)
                outs[b][bb, pl.ds(g * 32 + 16, 16)] = (
                    his[0] * (1.0 / NUM_NEIGHBORS))

    for b in range(_NBUF):
        @pl.when(b < n_chunks)
        def _(b=b):
            start(b, b)

    def step(c, _):
        b = lax.rem(c, _NBUF)
        for bs in range(_NBUF):
            @pl.when(b == bs)
            def _(bs=bs):
                pltpu.make_async_copy(theta_hbm.at[idx_v.at[c]], bufs[bs],
                                      gsems[bs]).wait()

                @pl.when(c >= _NBUF)
                def _():
                    pltpu.make_async_copy(
                        outs[bs],
                        out_hbm.at[pl.ds(0, _B_PER_CHUNK)],
                        ssems[bs]).wait()

                reduce(bs)
                pltpu.async_copy(
                    outs[bs],
                    out_hbm.at[pl.ds((start_chunk + c) * _B_PER_CHUNK,
                                     _B_PER_CHUNK)],
                    ssems[bs])

                @pl.when(c + _NBUF < n_chunks)
                def _():
                    start(c + _NBUF, bs)
        return ()

    lax.fori_loop(0, n_chunks, step, (), unroll=False)
    for b in range(_NBUF):
        @pl.when(b < n_chunks)
        def _(b=b):
            pltpu.make_async_copy(outs[b],
                                  out_hbm.at[pl.ds(0, _B_PER_CHUNK)],
                                  ssems[b]).wait()


@jax.jit
def kernel(node_ids, neighbor_lists, theta):
    del node_ids
    theta_bf = theta.astype(jnp.bfloat16)[:, _COL_PERM]
    idx = neighbor_lists.reshape(_N_CHUNKS, _CHUNK_IDX)

    mesh = plsc.VectorSubcoreMesh(core_axis_name="c", subcore_axis_name="s")
    out = pl.kernel(
        _tec_body,
        out_type=jax.ShapeDtypeStruct((BATCH, EMBED_DIM), jnp.float32),
        mesh=mesh,
        compiler_params=pltpu.CompilerParams(needs_layout_passes=False,
                                             use_tc_tiling_on_sc=False),
        scratch_types=[
            pltpu.VMEM((_CHUNKS_LO + 1, _CHUNK_IDX), jnp.int32),
            pltpu.VMEM((_CHUNK_IDX, EMBED_DIM), jnp.bfloat16),
            pltpu.VMEM((_CHUNK_IDX, EMBED_DIM), jnp.bfloat16),
            pltpu.VMEM((_B_PER_CHUNK, EMBED_DIM), jnp.float32),
            pltpu.VMEM((_B_PER_CHUNK, EMBED_DIM), jnp.float32),
            pltpu.SemaphoreType.DMA,
            pltpu.SemaphoreType.DMA,
            pltpu.SemaphoreType.DMA,
            pltpu.SemaphoreType.DMA,
        ],
    )(theta_bf, idx)
    return out


# speedup vs baseline: 3.7731x; 1.0708x over previous
import functools
import numpy as np
import jax
import jax.numpy as jnp
from jax import lax
from jax.experimental import pallas as pl
from jax.experimental.pallas import tpu as pltpu
from jax.experimental.pallas import tpu_sc as plsc

NUM_NODES = 100000
EMBED_DIM = 128
BATCH = 10000
NUM_NEIGHBORS = 32

_NC, _NS = 2, 16
_NW = _NC * _NS
_CHUNK_IDX = 128
_B_PER_CHUNK = _CHUNK_IDX // NUM_NEIGHBORS
_N_CHUNKS = BATCH * NUM_NEIGHBORS // _CHUNK_IDX
_CHUNKS_LO = _N_CHUNKS // _NW
_N_HI = _N_CHUNKS - _CHUNKS_LO * _NW
_NBUF = 2

def _tec_body(theta_hbm, idx_hbm, out_hbm, idx_v, rows0, rows1, oc0, oc1,
              gsem0, gsem1, ssem0, ssem1):
    wid = lax.axis_index("s") * _NC + lax.axis_index("c")
    is_hi = wid < _N_HI
    start_chunk = jnp.where(is_hi, (_CHUNKS_LO + 1) * wid,
                            (_CHUNKS_LO + 1) * _N_HI
                            + _CHUNKS_LO * (wid - _N_HI))
    n_chunks = jnp.where(is_hi, _CHUNKS_LO + 1, _CHUNKS_LO)

    pltpu.sync_copy(idx_hbm.at[pl.ds(start_chunk, _CHUNKS_LO)],
                    idx_v.at[pl.ds(0, _CHUNKS_LO)])

    @pl.when(is_hi)
    def _():
        pltpu.sync_copy(idx_hbm.at[pl.ds(start_chunk + _CHUNKS_LO, 1)],
                        idx_v.at[pl.ds(_CHUNKS_LO, 1)])

    bufs = (rows0, rows1)
    outs = (oc0, oc1)
    gsems = (gsem0, gsem1)
    ssems = (ssem0, ssem1)

    def start(c, b):
        pltpu.async_copy(theta_hbm.at[idx_v.at[c]], bufs[b], gsems[b])

    def reduce(b):
        rows = bufs[b]
        for bb in range(_B_PER_CHUNK):
            for d in range(EMBED_DIM // 16):
                sl = pl.ds(d * 16, 16)
                vals = [rows[bb * NUM_NEIGHBORS + k, sl]
                        for k in range(NUM_NEIGHBORS)]
                while len(vals) > 1:
                    vals = [vals[i] + vals[i + 1] for i in range(0, len(vals), 2)]
                outs[b][bb, sl] = vals[0] * (1.0 / NUM_NEIGHBORS)

    for b in range(_NBUF):
        @pl.when(b < n_chunks)
        def _(b=b):
            start(b, b)

    def step(c, _):
        b = lax.rem(c, _NBUF)
        for bs in range(_NBUF):
            @pl.when(b == bs)
            def _(bs=bs):
                pltpu.make_async_copy(theta_hbm.at[idx_v.at[c]], bufs[bs],
                                      gsems[bs]).wait()

                @pl.when(c >= _NBUF)
                def _():
                    pltpu.make_async_copy(
                        outs[bs],
                        out_hbm.at[pl.ds(0, _B_PER_CHUNK)],
                        ssems[bs]).wait()

                reduce(bs)
                pltpu.async_copy(
                    outs[bs],
                    out_hbm.at[pl.ds((start_chunk + c) * _B_PER_CHUNK,
                                     _B_PER_CHUNK)],
                    ssems[bs])

                @pl.when(c + _NBUF < n_chunks)
                def _():
                    start(c + _NBUF, bs)
        return ()

    lax.fori_loop(0, n_chunks, step, (), unroll=False)
    for b in range(_NBUF):
        @pl.when(b < n_chunks)
        def _(b=b):
            pltpu.make_async_copy(outs[b],
                                  out_hbm.at[pl.ds(0, _B_PER_CHUNK)],
                                  ssems[b]).wait()


@jax.jit
def kernel(node_ids, neighbor_lists, theta):
    del node_ids
    idx = neighbor_lists.reshape(_N_CHUNKS, _CHUNK_IDX)

    mesh = plsc.VectorSubcoreMesh(core_axis_name="c", subcore_axis_name="s")
    out = pl.kernel(
        _tec_body,
        out_type=jax.ShapeDtypeStruct((BATCH, EMBED_DIM), jnp.float32),
        mesh=mesh,
        compiler_params=pltpu.CompilerParams(needs_layout_passes=False,
                                             use_tc_tiling_on_sc=False),
        scratch_types=[
            pltpu.VMEM((_CHUNKS_LO + 1, _CHUNK_IDX), jnp.int32),
            pltpu.VMEM((_CHUNK_IDX, EMBED_DIM), jnp.float32),
            pltpu.VMEM((_CHUNK_IDX, EMBED_DIM), jnp.float32),
            pltpu.VMEM((_B_PER_CHUNK, EMBED_DIM), jnp.float32),
            pltpu.VMEM((_B_PER_CHUNK, EMBED_DIM), jnp.float32),
            pltpu.SemaphoreType.DMA,
            pltpu.SemaphoreType.DMA,
            pltpu.SemaphoreType.DMA,
            pltpu.SemaphoreType.DMA,
        ],
    )(theta, idx)
    return out
